# baseline (speedup 1.0000x reference)
import functools

import jax
import jax.numpy as jnp
from jax import lax
from jax.experimental import pallas as pl
from jax.experimental.pallas import tpu as pltpu
from jax.experimental.pallas import tpu_sc as plsc


def _pack_kernel(nf_ref, w_ref, m_ref, r_ref):
    w = w_ref[...]
    b_dim = nf_ref.shape[0]
    d_idx = jax.lax.broadcasted_iota(jnp.int32, (6, 2), 0)
    col = jax.lax.broadcasted_iota(jnp.int32, (6, 2), 1)
    shift = jnp.where(col == 0, d_idx, (d_idx + 3) % 6)
    wmr = (1 << shift).astype(jnp.float32)
    for b in range(b_dim):
        act = jnp.dot(nf_ref[b], w, preferred_element_type=jnp.float32)
        phtf = (act > 0).astype(jnp.float32)
        pk = jnp.dot(phtf, wmr, preferred_element_type=jnp.float32)
        pk = pk.astype(jnp.int32)
        m_ref[b] = jnp.transpose(pk[:, 0:1])
        r_ref[b] = jnp.transpose(pk[:, 1:2])


def _make_sc_main(B, N):
    NW = 32
    RPW = N // NW
    L = 16
    NCHUNK = N // L

    mesh = plsc.VectorSubcoreMesh(core_axis_name="c", subcore_axis_name="s")

    @functools.partial(
        pl.kernel,
        mesh=mesh,
        out_type=jax.ShapeDtypeStruct((B, N, N), jnp.float32),
        scratch_types=[
            pltpu.VMEM((2, N), jnp.int32),
            pltpu.VMEM((2, B, N), jnp.float32),
            pltpu.VMEM((B, N), jnp.int32),
            pltpu.VMEM((B * RPW,), jnp.int32),
            pltpu.SemaphoreType.DMA,
            pltpu.SemaphoreType.DMA,
        ],
    )
    def sc_main(dir_hbm, m_hbm, r_hbm, out_hbm, dirbuf, outbuf, rbuf, mbuf, sem_d, sem_o):
        wid = lax.axis_index("s") * 2 + lax.axis_index("c")
        base = wid * RPW
        pltpu.sync_copy(r_hbm.at[:, 0, :], rbuf)
        for b in range(B):
            pltpu.sync_copy(m_hbm.at[b, 0, pl.ds(base, RPW)],
                            mbuf.at[pl.ds(b * RPW, RPW)])

        def dir_copy(i, slot):
            return pltpu.make_async_copy(
                dir_hbm.at[base + i, :], dirbuf.at[slot], sem_d)

        def out_copy(i, slot, b):
            return pltpu.make_async_copy(
                outbuf.at[slot, b], out_hbm.at[b, base + i, :], sem_o)

        dir_copy(0, 0).start()

        def row_body(i, _):
            slot = i % 2

            @pl.when(i + 1 < RPW)
            def _prefetch():
                dir_copy(i + 1, (i + 1) % 2).start()

            dir_copy(i, slot).wait()

            @pl.when(i >= 2)
            def _drain():
                for b in range(B):
                    out_copy(i - 2, slot, b).wait()

            for b in range(B):
                flat = b * RPW + i
                chunk = mbuf[pl.ds((flat // L) * L, L)]
                mb = lax.gather(
                    chunk,
                    jnp.full((L, 1), flat % L, jnp.int32),
                    lax.GatherDimensionNumbers(
                        offset_dims=(), collapsed_slice_dims=(0,),
                        start_index_map=(0,)),
                    slice_sizes=(1,),
                    mode=lax.GatherScatterMode.PROMISE_IN_BOUNDS)

                def body(c, _, b=b, slot=slot, mb=mb):
                    sl = pl.ds(c * L, L)
                    d = dirbuf[slot, sl]
                    rr = rbuf[b, sl]
                    outbuf[slot, b, sl] = (((mb & rr) >> d) & 1).astype(jnp.float32)
                    return 0

                lax.fori_loop(0, NCHUNK, body, 0, unroll=4)

            for b in range(B):
                out_copy(i, slot, b).start()
            return 0

        lax.fori_loop(0, RPW, row_body, 0)
        for i in (RPW - 2, RPW - 1):
            for b in range(B):
                out_copy(i, i % 2, b).wait()

    return sc_main


@functools.partial(jax.jit, static_argnames=())
def kernel(node_features, direction_matrix, port_feature_mask):
    B, N, F = node_features.shape
    dir32 = direction_matrix.astype(jnp.int32)

    m, r = pl.pallas_call(
        _pack_kernel,
        out_shape=(
            jax.ShapeDtypeStruct((B, 1, N), jnp.int32),
            jax.ShapeDtypeStruct((B, 1, N), jnp.int32),
        ),
    )(node_features, port_feature_mask)

    out = _make_sc_main(B, N)(dir32, m, r)
    return out


# speedup vs baseline: 1.0268x; 1.0268x over previous
import functools

import jax
import jax.numpy as jnp
from jax import lax
from jax.experimental import pallas as pl
from jax.experimental.pallas import tpu as pltpu
from jax.experimental.pallas import tpu_sc as plsc


def _pack_kernel(nf_ref, w_ref, m_ref, r_ref):
    w = w_ref[...]
    b_dim = nf_ref.shape[0]
    d_idx = jax.lax.broadcasted_iota(jnp.int32, (6, 2), 0)
    col = jax.lax.broadcasted_iota(jnp.int32, (6, 2), 1)
    shift = jnp.where(col == 0, d_idx, (d_idx + 3) % 6)
    wmr = (1 << shift).astype(jnp.float32)
    for b in range(b_dim):
        act = jnp.dot(nf_ref[b], w, preferred_element_type=jnp.float32)
        phtf = (act > 0).astype(jnp.float32)
        pk = jnp.dot(phtf, wmr, preferred_element_type=jnp.float32)
        pk = pk.astype(jnp.int32)
        m_ref[b] = jnp.transpose(pk[:, 0:1])
        r_ref[b] = jnp.transpose(pk[:, 1:2])


def _make_sc_main(B, N):
    NW = 32
    RPW = N // NW
    L = 16
    NCHUNK = N // L

    mesh = plsc.VectorSubcoreMesh(core_axis_name="c", subcore_axis_name="s")

    @functools.partial(
        pl.kernel,
        mesh=mesh,
        out_type=jax.ShapeDtypeStruct((B, N, N), jnp.float32),
        scratch_types=[
            pltpu.VMEM((2, N), jnp.int32),
            pltpu.VMEM((2, B, N), jnp.float32),
            pltpu.VMEM((B, N), jnp.int32),
            pltpu.VMEM((B * RPW,), jnp.int32),
            pltpu.SemaphoreType.DMA,
            pltpu.SemaphoreType.DMA,
        ],
    )
    def sc_main(dir_hbm, m_hbm, r_hbm, out_hbm, dirbuf, outbuf, rbuf, mbuf, sem_d, sem_o):
        wid = lax.axis_index("s") * 2 + lax.axis_index("c")
        base = wid * RPW
        pltpu.sync_copy(r_hbm.at[:, 0, :], rbuf)
        for b in range(B):
            pltpu.sync_copy(m_hbm.at[b, 0, pl.ds(base, RPW)],
                            mbuf.at[pl.ds(b * RPW, RPW)])

        def dir_copy(i, slot):
            return pltpu.make_async_copy(
                dir_hbm.at[base + i, :], dirbuf.at[slot], sem_d)

        def out_copy(i, slot, b):
            return pltpu.make_async_copy(
                outbuf.at[slot, b], out_hbm.at[b, base + i, :], sem_o)

        dir_copy(0, 0).start()

        def row_body(i, _):
            slot = i % 2

            @pl.when(i + 1 < RPW)
            def _prefetch():
                dir_copy(i + 1, (i + 1) % 2).start()

            dir_copy(i, slot).wait()

            @pl.when(i >= 2)
            def _drain():
                for b in range(B):
                    out_copy(i - 2, slot, b).wait()

            mbs = []
            for b in range(B):
                flat = b * RPW + i
                chunk = mbuf[pl.ds((flat // L) * L, L)]
                mbs.append(lax.gather(
                    chunk,
                    jnp.full((L, 1), flat % L, jnp.int32),
                    lax.GatherDimensionNumbers(
                        offset_dims=(), collapsed_slice_dims=(0,),
                        start_index_map=(0,)),
                    slice_sizes=(1,),
                    mode=lax.GatherScatterMode.PROMISE_IN_BOUNDS))

            def body(c, _, slot=slot, mbs=mbs):
                sl = pl.ds(c * L, L)
                d = dirbuf[slot, sl]
                for b in range(B):
                    rr = rbuf[b, sl]
                    outbuf[slot, b, sl] = (((mbs[b] & rr) >> d) & 1).astype(jnp.float32)
                return 0

            lax.fori_loop(0, NCHUNK, body, 0, unroll=4)

            for b in range(B):
                out_copy(i, slot, b).start()
            return 0

        lax.fori_loop(0, RPW, row_body, 0)
        for i in (RPW - 2, RPW - 1):
            for b in range(B):
                out_copy(i, i % 2, b).wait()

    return sc_main


@functools.partial(jax.jit, static_argnames=())
def kernel(node_features, direction_matrix, port_feature_mask):
    B, N, F = node_features.shape
    dir32 = direction_matrix.astype(jnp.int32)

    m, r = pl.pallas_call(
        _pack_kernel,
        out_shape=(
            jax.ShapeDtypeStruct((B, 1, N), jnp.int32),
            jax.ShapeDtypeStruct((B, 1, N), jnp.int32),
        ),
    )(node_features, port_feature_mask)

    out = _make_sc_main(B, N)(dir32, m, r)
    return out


# speedup vs baseline: 6.8975x; 6.7176x over previous
import functools

import jax
import jax.numpy as jnp
from jax.experimental import pallas as pl
from jax.experimental.pallas import tpu as pltpu


def _fused_kernel(nf_ref, w_ref, dir_ref, out_ref, m_ref, r_ref):
    i = pl.program_id(0)
    b_dim = out_ref.shape[0]
    bi = dir_ref.shape[0]

    n = r_ref.shape[-1]

    @pl.when(i == 0)
    def _pack():
        w = w_ref[...]
        d_idx = jax.lax.broadcasted_iota(jnp.int32, (6, 2), 0)
        col = jax.lax.broadcasted_iota(jnp.int32, (6, 2), 1)
        shift = jnp.where(col == 0, d_idx, (d_idx + 3) % 6)
        wmr = (1 << shift).astype(jnp.float32)
        for b in range(b_dim):
            act = jnp.dot(nf_ref[b], w, preferred_element_type=jnp.float32)
            phtf = (act > 0).astype(jnp.float32)
            pk = jnp.dot(phtf, wmr, preferred_element_type=jnp.float32)
            pk = pk.astype(jnp.int32)
            m_ref[b] = pk[:, 0:1]
            r_ref[b] = jnp.transpose(pk[:, 1:2])

    d = dir_ref[...]
    for b in range(b_dim):
        mb = m_ref[b, pl.ds(i * bi, bi), :]
        rb = r_ref[b]
        combined = mb & rb
        out_ref[b] = ((combined >> d) & 1).astype(jnp.float32)


@functools.partial(jax.jit, static_argnames=())
def kernel(node_features, direction_matrix, port_feature_mask):
    B, N, F = node_features.shape
    dir32 = direction_matrix.astype(jnp.int32)

    BI = 256
    grid = (N // BI,)
    out = pl.pallas_call(
        _fused_kernel,
        grid=grid,
        in_specs=[
            pl.BlockSpec((B, N, F), lambda i: (0, 0, 0)),
            pl.BlockSpec((F, 6), lambda i: (0, 0)),
            pl.BlockSpec((BI, N), lambda i: (i, 0)),
        ],
        out_specs=pl.BlockSpec((B, BI, N), lambda i: (0, i, 0)),
        out_shape=jax.ShapeDtypeStruct((B, N, N), jnp.float32),
        scratch_shapes=[
            pltpu.VMEM((B, N, 1), jnp.int32),
            pltpu.VMEM((B, 1, N), jnp.int32),
        ],
    )(node_features, port_feature_mask, dir32)
    return out


# speedup vs baseline: 6.9659x; 1.0099x over previous
import functools

import jax
import jax.numpy as jnp
from jax.experimental import pallas as pl
from jax.experimental.pallas import tpu as pltpu


def _fused_kernel(nf_ref, w_ref, dir_ref, out_ref, m_ref, r_ref):
    i = pl.program_id(0)
    b_dim = out_ref.shape[0]
    bi = dir_ref.shape[0]

    n = r_ref.shape[-1]

    @pl.when(i == 0)
    def _pack():
        w = w_ref[...]
        d_idx = jax.lax.broadcasted_iota(jnp.int32, (6, 2), 0)
        col = jax.lax.broadcasted_iota(jnp.int32, (6, 2), 1)
        shift = jnp.where(col == 0, d_idx, (d_idx + 3) % 6)
        wmr = (1 << shift).astype(jnp.float32)
        for b in range(b_dim):
            act = jnp.dot(nf_ref[b], w, preferred_element_type=jnp.float32)
            phtf = (act > 0).astype(jnp.float32)
            pk = jnp.dot(phtf, wmr, preferred_element_type=jnp.float32)
            pk = pk.astype(jnp.int32)
            m_ref[b] = pk[:, 0:1]
            r_ref[b] = jnp.transpose(pk[:, 1:2])

    d = dir_ref[...]
    for b in range(b_dim):
        mb = m_ref[b, pl.ds(i * bi, bi), :]
        rb = r_ref[b]
        combined = mb & rb
        out_ref[b] = ((combined >> d) & 1).astype(jnp.float32)


@functools.partial(jax.jit, static_argnames=())
def kernel(node_features, direction_matrix, port_feature_mask):
    B, N, F = node_features.shape
    dir32 = direction_matrix.astype(jnp.int32)

    BI = 512
    grid = (N // BI,)
    out = pl.pallas_call(
        _fused_kernel,
        grid=grid,
        in_specs=[
            pl.BlockSpec((B, N, F), lambda i: (0, 0, 0)),
            pl.BlockSpec((F, 6), lambda i: (0, 0)),
            pl.BlockSpec((BI, N), lambda i: (i, 0)),
        ],
        out_specs=pl.BlockSpec((B, BI, N), lambda i: (0, i, 0)),
        out_shape=jax.ShapeDtypeStruct((B, N, N), jnp.float32),
        scratch_shapes=[
            pltpu.VMEM((B, N, 1), jnp.int32),
            pltpu.VMEM((B, 1, N), jnp.int32),
        ],
    )(node_features, port_feature_mask, dir32)
    return out
